# baseline (speedup 1.0000x reference)
import jax
import jax.numpy as jnp
from jax import lax
from jax.experimental import pallas as pl
from jax.experimental.pallas import tpu as pltpu
from jax.experimental.pallas import tpu_sc as plsc

B, S, D = 4, 4096, 768
N = B * S
NC, NS = 2, 16
NW = NC * NS
NH = 2
DH = D // NH
NG = NW // NH
TPG = N // NG
T = 16
NBT = TPG // T
LANES = 16
DV = DH // LANES


def _sc_body(x_hbm, pos_hbm, vidx_hbm, lidx_hbm, var_hbm, time_hbm,
             out_hbm, vidx_all, lidx_all,
             xb0, pb0, ob0, xb1, pb1, ob1, var_t, time_t,
             sem_in0, sem_in1, sem_out0, sem_out1):
  wid = lax.axis_index("s") * NC + lax.axis_index("c")
  g = wid // NH
  h = wid % NH
  tok0 = pl.multiple_of(g * TPG, TPG)
  col0 = pl.multiple_of(h * DH, DH)
  cols = pl.ds(col0, DH)

  pltpu.sync_copy(var_hbm.at[:, cols], var_t)
  pltpu.sync_copy(time_hbm.at[:, cols], time_t)
  pltpu.sync_copy(vidx_hbm.at[pl.ds(tok0, TPG)], vidx_all)
  pltpu.sync_copy(lidx_hbm.at[pl.ds(tok0, TPG)], lidx_all)

  bufs = ((xb0, pb0, ob0, sem_in0, sem_out0),
          (xb1, pb1, ob1, sem_in1, sem_out1))

  def rows(blk):
    return pl.ds(tok0 + blk * T, T)

  def fire_in(blk, xb, pb, sem):
    pltpu.async_copy(x_hbm.at[rows(blk), cols], xb, sem)
    pltpu.async_copy(pos_hbm.at[rows(blk), cols], pb, sem)

  dios = [jnp.arange(dv * LANES, (dv + 1) * LANES, dtype=jnp.int32)
          for dv in range(DV)]

  dnums = lax.GatherDimensionNumbers(
      offset_dims=(), collapsed_slice_dims=(0,), start_index_map=(0,))

  def lane_bcast(vec, lane):
    return lax.gather(vec, lane[:, None], dnums, (1,),
                      mode=lax.GatherScatterMode.PROMISE_IN_BOUNDS)

  def half(parity, blk):
    xb, pb, ob, sem_in, sem_out = bufs[parity]
    pltpu.make_async_copy(x_hbm.at[rows(blk), cols], xb, sem_in).wait()
    pltpu.make_async_copy(pos_hbm.at[rows(blk), cols], pb, sem_in).wait()

    @pl.when(blk >= 2)
    def _():
      pltpu.make_async_copy(ob, out_hbm.at[rows(blk), cols], sem_out).wait()

    seg = pl.ds(pl.multiple_of(blk * T, T), LANES)
    vsegv = vidx_all[seg]
    lsegv = lidx_all[seg]

    @plsc.parallel_loop(0, T, unroll=2)
    def token_step(t):
      lane = jnp.broadcast_to(t, (LANES,))
      rv = lane_bcast(vsegv, lane)
      rt = lane_bcast(lsegv, lane)
      for dv in range(DV):
        s = pl.ds(dv * LANES, LANES)
        varv = plsc.load_gather(var_t, [rv, dios[dv]])
        timv = plsc.load_gather(time_t, [rt, dios[dv]])
        ob[t, s] = xb[t, s] + pb[t, s] + varv + timv

    pltpu.async_copy(ob, out_hbm.at[rows(blk), cols], sem_out)

    @pl.when(blk + 2 < NBT)
    def _():
      fire_in(blk + 2, xb, pb, sem_in)

  fire_in(0, xb0, pb0, sem_in0)
  fire_in(1, xb1, pb1, sem_in1)

  def pair(gg, _):
    half(0, gg * 2)
    half(1, gg * 2 + 1)
    return 0

  lax.fori_loop(0, NBT // 2, pair, 0)
  pltpu.make_async_copy(ob0, out_hbm.at[rows(0), cols], sem_out0).wait()
  pltpu.make_async_copy(ob1, out_hbm.at[rows(1), cols], sem_out1).wait()


@jax.jit
def _sc_embed(x2, pos2, vidx, lidx, var_table, time_table):
  mesh = plsc.VectorSubcoreMesh(
      core_axis_name="c", subcore_axis_name="s",
      num_cores=NC, num_subcores=NS)
  return pl.kernel(
      _sc_body,
      out_type=jax.ShapeDtypeStruct((N, D), jnp.float32),
      mesh=mesh,
      compiler_params=pltpu.CompilerParams(needs_layout_passes=False),
      scratch_types=[
          pltpu.VMEM((TPG,), jnp.int32),
          pltpu.VMEM((TPG,), jnp.int32),
          pltpu.VMEM((T, DH), jnp.float32),
          pltpu.VMEM((T, DH), jnp.float32),
          pltpu.VMEM((T, DH), jnp.float32),
          pltpu.VMEM((T, DH), jnp.float32),
          pltpu.VMEM((T, DH), jnp.float32),
          pltpu.VMEM((T, DH), jnp.float32),
          pltpu.VMEM((100, DH), jnp.float32),
          pltpu.VMEM((100, DH), jnp.float32),
          pltpu.SemaphoreType.DMA,
          pltpu.SemaphoreType.DMA,
          pltpu.SemaphoreType.DMA,
          pltpu.SemaphoreType.DMA,
      ],
  )(x2, pos2, vidx, lidx, var_table, time_table)


def kernel(x, variable_seq, pos_emb, lead_time_seq, var_table, time_table):
  x2 = x.reshape(N, D)
  pos2 = pos_emb.reshape(N, D)
  vidx = variable_seq.reshape(N).astype(jnp.int32)
  lidx = lead_time_seq.reshape(N).astype(jnp.int32)
  out = _sc_embed(x2, pos2, vidx, lidx, var_table, time_table)
  return out.reshape(B, S, D)


# speedup vs baseline: 1.3392x; 1.3392x over previous
import jax
import jax.numpy as jnp
from jax import lax
from jax.experimental import pallas as pl
from jax.experimental.pallas import tpu as pltpu
from jax.experimental.pallas import tpu_sc as plsc

B, S, D = 4, 4096, 768
N = B * S
NC, NS = 2, 16
NW = NC * NS
NH = 2
DH = D // NH
NG = NW // NH
TPG = N // NG
T = 16
NBT = TPG // T
LANES = 16
DV = DH // LANES


def _sc_body(x_hbm, pos_hbm, vidx_hbm, lidx_hbm, var_hbm, time_hbm,
             out_hbm, vidx_all, lidx_all,
             xb0, pb0, ob0, xb1, pb1, ob1, var_t, time_t,
             sem_in0, sem_in1, sem_out0, sem_out1):
  wid = lax.axis_index("s") * NC + lax.axis_index("c")
  g = wid // NH
  h = wid % NH
  tok0 = pl.multiple_of(g * TPG, TPG)
  col0 = pl.multiple_of(h * DH, DH)
  cols = pl.ds(col0, DH)

  pltpu.sync_copy(var_hbm.at[h], var_t)
  pltpu.sync_copy(time_hbm.at[h], time_t)
  pltpu.sync_copy(vidx_hbm.at[pl.ds(tok0, TPG)], vidx_all)
  pltpu.sync_copy(lidx_hbm.at[pl.ds(tok0, TPG)], lidx_all)

  bufs = ((xb0, pb0, ob0, sem_in0, sem_out0),
          (xb1, pb1, ob1, sem_in1, sem_out1))

  def rows(blk):
    return pl.ds(tok0 + blk * T, T)

  def fire_in(blk, xb, pb, sem):
    pltpu.async_copy(x_hbm.at[rows(blk), cols], xb, sem)
    pltpu.async_copy(pos_hbm.at[rows(blk), cols], pb, sem)

  dios = [jnp.arange(dv * LANES, (dv + 1) * LANES, dtype=jnp.int32)
          for dv in range(DV)]

  dnums = lax.GatherDimensionNumbers(
      offset_dims=(), collapsed_slice_dims=(0,), start_index_map=(0,))

  def lane_bcast(vec, lane):
    return lax.gather(vec, lane[:, None], dnums, (1,),
                      mode=lax.GatherScatterMode.PROMISE_IN_BOUNDS)

  def half(parity, blk):
    xb, pb, ob, sem_in, sem_out = bufs[parity]
    pltpu.make_async_copy(x_hbm.at[rows(blk), cols], xb, sem_in).wait()
    pltpu.make_async_copy(pos_hbm.at[rows(blk), cols], pb, sem_in).wait()

    @pl.when(blk >= 2)
    def _():
      pltpu.make_async_copy(ob, out_hbm.at[rows(blk), cols], sem_out).wait()

    seg = pl.ds(pl.multiple_of(blk * T, T), LANES)
    vsegv = vidx_all[seg]
    lsegv = lidx_all[seg]

    @plsc.parallel_loop(0, T)
    def token_step(t):
      lane = jnp.broadcast_to(t, (LANES,))
      rv = lane_bcast(vsegv, lane) * DH
      rt = lane_bcast(lsegv, lane) * DH
      for dv in range(DV):
        s = pl.ds(dv * LANES, LANES)
        varv = plsc.load_gather(var_t, [rv + dios[dv]])
        timv = plsc.load_gather(time_t, [rt + dios[dv]])
        ob[t, s] = xb[t, s] + pb[t, s] + varv + timv

    pltpu.async_copy(ob, out_hbm.at[rows(blk), cols], sem_out)

    @pl.when(blk + 2 < NBT)
    def _():
      fire_in(blk + 2, xb, pb, sem_in)

  fire_in(0, xb0, pb0, sem_in0)
  fire_in(1, xb1, pb1, sem_in1)

  def pair(gg, _):
    half(0, gg * 2)
    half(1, gg * 2 + 1)
    return 0

  lax.fori_loop(0, NBT // 2, pair, 0)
  pltpu.make_async_copy(ob0, out_hbm.at[rows(0), cols], sem_out0).wait()
  pltpu.make_async_copy(ob1, out_hbm.at[rows(1), cols], sem_out1).wait()


@jax.jit
def _sc_embed(x2, pos2, vidx, lidx, var_table, time_table):
  mesh = plsc.VectorSubcoreMesh(
      core_axis_name="c", subcore_axis_name="s",
      num_cores=NC, num_subcores=NS)
  return pl.kernel(
      _sc_body,
      out_type=jax.ShapeDtypeStruct((N, D), jnp.float32),
      mesh=mesh,
      compiler_params=pltpu.CompilerParams(needs_layout_passes=False),
      scratch_types=[
          pltpu.VMEM((TPG,), jnp.int32),
          pltpu.VMEM((TPG,), jnp.int32),
          pltpu.VMEM((T, DH), jnp.float32),
          pltpu.VMEM((T, DH), jnp.float32),
          pltpu.VMEM((T, DH), jnp.float32),
          pltpu.VMEM((T, DH), jnp.float32),
          pltpu.VMEM((T, DH), jnp.float32),
          pltpu.VMEM((T, DH), jnp.float32),
          pltpu.VMEM((100 * DH,), jnp.float32),
          pltpu.VMEM((100 * DH,), jnp.float32),
          pltpu.SemaphoreType.DMA,
          pltpu.SemaphoreType.DMA,
          pltpu.SemaphoreType.DMA,
          pltpu.SemaphoreType.DMA,
      ],
  )(x2, pos2, vidx, lidx, var_table, time_table)


def kernel(x, variable_seq, pos_emb, lead_time_seq, var_table, time_table):
  x2 = x.reshape(N, D)
  pos2 = pos_emb.reshape(N, D)
  vidx = variable_seq.reshape(N).astype(jnp.int32)
  lidx = lead_time_seq.reshape(N).astype(jnp.int32)
  var_r = var_table.reshape(100, NH, DH).transpose(1, 0, 2).reshape(NH, 100 * DH)
  time_r = time_table.reshape(100, NH, DH).transpose(1, 0, 2).reshape(NH, 100 * DH)
  out = _sc_embed(x2, pos2, vidx, lidx, var_r, time_r)
  return out.reshape(B, S, D)


# speedup vs baseline: 1.6026x; 1.1967x over previous
import jax
import jax.numpy as jnp
from jax import lax
from jax.experimental import pallas as pl
from jax.experimental.pallas import tpu as pltpu
from jax.experimental.pallas import tpu_sc as plsc

B, S, D = 4, 4096, 768
N = B * S
NC, NS = 2, 16
NW = NC * NS
NH = 2
DH = D // NH
NG = NW // NH
TPG = N // NG
T = 32
NBT = TPG // T
LANES = 16
WPT = DH // 2
NGRP = DH // 32
V = 100


def _sc_body(x_hbm, pos_hbm, vidx_hbm, lidx_hbm, var_hbm, time_hbm,
             out_hbm, vidx_all, lidx_all,
             xb0, pb0, ob0, xb1, pb1, ob1, var_t, time_t,
             sem_in0, sem_in1, sem_out0, sem_out1):
  wid = lax.axis_index("s") * NC + lax.axis_index("c")
  g = wid // NH
  h = wid % NH
  tok0 = pl.multiple_of(g * TPG, TPG)
  col0 = pl.multiple_of(h * DH, DH)
  cols = pl.ds(col0, DH)

  pltpu.sync_copy(var_hbm.at[h], var_t)
  pltpu.sync_copy(time_hbm.at[h], time_t)
  pltpu.sync_copy(vidx_hbm.at[pl.ds(tok0, TPG)], vidx_all)
  pltpu.sync_copy(lidx_hbm.at[pl.ds(tok0, TPG)], lidx_all)

  bufs = ((xb0, pb0, ob0, sem_in0, sem_out0),
          (xb1, pb1, ob1, sem_in1, sem_out1))

  def rows(blk):
    return pl.ds(tok0 + blk * T, T)

  def fire_in(blk, xb, pb, sem):
    pltpu.async_copy(x_hbm.at[rows(blk), cols], xb, sem)
    pltpu.async_copy(pos_hbm.at[rows(blk), cols], pb, sem)

  wios = [jnp.arange(gg * LANES, (gg + 1) * LANES, dtype=jnp.int32)
          for gg in range(NGRP)]

  dnums = lax.GatherDimensionNumbers(
      offset_dims=(), collapsed_slice_dims=(0,), start_index_map=(0,))

  def lane_bcast(vec, lane):
    return lax.gather(vec, lane[:, None], dnums, (1,),
                      mode=lax.GatherScatterMode.PROMISE_IN_BOUNDS)

  def half(parity, blk):
    xb, pb, ob, sem_in, sem_out = bufs[parity]
    pltpu.make_async_copy(x_hbm.at[rows(blk), cols], xb, sem_in).wait()
    pltpu.make_async_copy(pos_hbm.at[rows(blk), cols], pb, sem_in).wait()

    @pl.when(blk >= 2)
    def _():
      pltpu.make_async_copy(ob, out_hbm.at[rows(blk), cols], sem_out).wait()

    @plsc.parallel_loop(0, T)
    def token_step(t):
      seg = pl.ds(blk * T + (t & ~(LANES - 1)), LANES)
      lane = jnp.broadcast_to(t & (LANES - 1), (LANES,))
      rvw = lane_bcast(vidx_all[seg], lane) * WPT
      rtw = lane_bcast(lidx_all[seg], lane) * WPT
      for gg in range(NGRP):
        wv = plsc.load_gather(var_t, [rvw + wios[gg]])
        wt = plsc.load_gather(time_t, [rtw + wios[gg]])
        va, vb = plsc.unpack(plsc.bitcast(wv, jnp.bfloat16),
                             format=plsc.PackFormat.INTERLEAVED)
        ta, tb = plsc.unpack(plsc.bitcast(wt, jnp.bfloat16),
                             format=plsc.PackFormat.INTERLEAVED)
        s0 = pl.ds(gg * 32, LANES)
        s1 = pl.ds(gg * 32 + LANES, LANES)
        ob[t, s0] = xb[t, s0] + pb[t, s0] + va + ta
        ob[t, s1] = xb[t, s1] + pb[t, s1] + vb + tb

    pltpu.async_copy(ob, out_hbm.at[rows(blk), cols], sem_out)

    @pl.when(blk + 2 < NBT)
    def _():
      fire_in(blk + 2, xb, pb, sem_in)

  fire_in(0, xb0, pb0, sem_in0)
  fire_in(1, xb1, pb1, sem_in1)

  def pair(gg, _):
    half(0, gg * 2)
    half(1, gg * 2 + 1)
    return 0

  lax.fori_loop(0, NBT // 2, pair, 0)
  pltpu.make_async_copy(ob0, out_hbm.at[rows(0), cols], sem_out0).wait()
  pltpu.make_async_copy(ob1, out_hbm.at[rows(1), cols], sem_out1).wait()


@jax.jit
def _sc_embed(x2, pos2, vidx, lidx, var_w, time_w):
  mesh = plsc.VectorSubcoreMesh(
      core_axis_name="c", subcore_axis_name="s",
      num_cores=NC, num_subcores=NS)
  return pl.kernel(
      _sc_body,
      out_type=jax.ShapeDtypeStruct((N, D), jnp.float32),
      mesh=mesh,
      compiler_params=pltpu.CompilerParams(needs_layout_passes=False),
      scratch_types=[
          pltpu.VMEM((TPG,), jnp.int32),
          pltpu.VMEM((TPG,), jnp.int32),
          pltpu.VMEM((T, DH), jnp.float32),
          pltpu.VMEM((T, DH), jnp.float32),
          pltpu.VMEM((T, DH), jnp.float32),
          pltpu.VMEM((T, DH), jnp.float32),
          pltpu.VMEM((T, DH), jnp.float32),
          pltpu.VMEM((T, DH), jnp.float32),
          pltpu.VMEM((V * WPT,), jnp.int32),
          pltpu.VMEM((V * WPT,), jnp.int32),
          pltpu.SemaphoreType.DMA,
          pltpu.SemaphoreType.DMA,
          pltpu.SemaphoreType.DMA,
          pltpu.SemaphoreType.DMA,
      ],
  )(x2, pos2, vidx, lidx, var_w, time_w)


def _pack_table(tab):
  tb = tab.astype(jnp.bfloat16)
  tb = tb.reshape(V, NH, NGRP, 2, LANES)
  tb = tb.transpose(1, 0, 2, 4, 3)
  words = lax.bitcast_convert_type(tb, jnp.int32)
  return words.reshape(NH, V * WPT)


def kernel(x, variable_seq, pos_emb, lead_time_seq, var_table, time_table):
  x2 = x.reshape(N, D)
  pos2 = pos_emb.reshape(N, D)
  vidx = variable_seq.reshape(N).astype(jnp.int32)
  lidx = lead_time_seq.reshape(N).astype(jnp.int32)
  out = _sc_embed(x2, pos2, vidx, lidx,
                  _pack_table(var_table), _pack_table(time_table))
  return out.reshape(B, S, D)
